# baseline (speedup 1.0000x reference)
import functools

import jax
import jax.numpy as jnp
from jax import lax
from jax.experimental import pallas as pl
from jax.experimental.pallas import tpu as pltpu
from jax.experimental.pallas import tpu_sc as plsc

_DIM = 32
_MBLK = 65536


def _row_mean_body(x_ref, o_ref):
    o_ref[...] = (jnp.sum(x_ref[...], axis=0) * (1.0 / _DIM)).astype(
        jnp.bfloat16)


def _row_means(table_t):
    rows = table_t.shape[1]
    grid = (rows + _MBLK - 1) // _MBLK
    return pl.pallas_call(
        _row_mean_body,
        grid=(grid,),
        in_specs=[pl.BlockSpec((_DIM, _MBLK), lambda i: (0, i))],
        out_specs=pl.BlockSpec((_MBLK,), lambda i: (i,)),
        out_shape=jax.ShapeDtypeStruct((grid * _MBLK,), jnp.bfloat16),
    )(table_t)


def _gather_means(means2, idx1d):
    info = plsc.get_sparse_core_info()
    nw = info.num_cores * info.num_subcores
    n = idx1d.shape[0]
    npw = n // nw
    m = means2.shape[0]
    mesh = plsc.VectorSubcoreMesh(core_axis_name="c", subcore_axis_name="s")

    @functools.partial(
        pl.kernel, mesh=mesh,
        out_type=pltpu.HBM((n,), jnp.int32),
        scratch_types=[
            pltpu.VMEM((npw,), jnp.int32),
            pltpu.VMEM((npw,), jnp.int32),
            pltpu.VMEM_SHARED((m,), jnp.int32),
            pltpu.SemaphoreType.DMA,
        ],
    )
    def gather_kernel(means_hbm, idx_hbm, out_hbm, idx_v, vals_v, shared,
                      sem):
        wid = lax.axis_index("s") * info.num_cores + lax.axis_index("c")
        base = wid * npw
        pltpu.sync_copy(idx_hbm.at[pl.ds(base, npw)], idx_v)

        @pl.when(lax.axis_index("s") == 0)
        def _stage():
            pltpu.sync_copy(means_hbm, shared)

        plsc.subcore_barrier()
        pltpu.async_copy(shared.at[idx_v], vals_v, sem).wait()
        pltpu.sync_copy(vals_v, out_hbm.at[pl.ds(base, npw)])

    return gather_kernel(means2, idx1d)


def _norm_body(aw_ref, pw_ref, nw_ref, ai_ref, pi_ref, ni_ref,
               oa_ref, op_ref, on_ref):
    for w_ref, i_ref, o_ref in ((aw_ref, ai_ref, oa_ref),
                                (pw_ref, pi_ref, op_ref),
                                (nw_ref, ni_ref, on_ref)):
        w = w_ref[...]
        odd = (i_ref[...] & 1) == 1
        half = jnp.where(odd, lax.shift_right_logical(w, 16), w)
        x = lax.bitcast_convert_type(lax.shift_left(half, 16), jnp.float32)
        ss = jnp.sum(x * x, axis=0, keepdims=True)
        o_ref[...] = x / jnp.sqrt(ss)


def _normalize(vw, vi):
    seq = vw[0].shape[0]
    ca, cn = vw[0].shape[2], vw[2].shape[2]
    grid = 8
    ba, bn = ca // grid, cn // grid
    spec_a = pl.BlockSpec((seq, 1, ba), lambda i: (0, 0, i))
    spec_n = pl.BlockSpec((seq, 1, bn), lambda i: (0, 0, i))
    return pl.pallas_call(
        _norm_body,
        grid=(grid,),
        in_specs=[spec_a, spec_a, spec_n, spec_a, spec_a, spec_n],
        out_specs=[spec_a, spec_a, spec_n],
        out_shape=[jax.ShapeDtypeStruct((seq, 1, ca), jnp.float32),
                   jax.ShapeDtypeStruct((seq, 1, ca), jnp.float32),
                   jax.ShapeDtypeStruct((seq, 1, cn), jnp.float32)],
    )(*vw, *vi)


def kernel(anchor_input_ids, positive_input_ids, negative_input_ids,
           embedding_table):
    batch, seq = anchor_input_ids.shape
    num_neg = negative_input_ids.shape[1]
    na = batch * seq

    means = _row_means(embedding_table.T)
    means2 = lax.bitcast_convert_type(
        means.reshape(-1, 2), jnp.int32)

    nt = negative_input_ids.transpose(2, 1, 0)
    nt = nt.reshape(seq, num_neg, batch // 128, 128).transpose(0, 2, 1, 3)
    ids = jnp.concatenate([
        anchor_input_ids.T.reshape(-1),
        positive_input_ids.T.reshape(-1),
        nt.reshape(-1),
    ]).astype(jnp.int32)
    words = _gather_means(means2, lax.shift_right_logical(ids, 1))

    vw = (words[:na].reshape(seq, 1, batch),
          words[na:2 * na].reshape(seq, 1, batch),
          words[2 * na:].reshape(seq, 1, num_neg * batch))
    vi = (ids[:na].reshape(seq, 1, batch),
          ids[na:2 * na].reshape(seq, 1, batch),
          ids[2 * na:].reshape(seq, 1, num_neg * batch))
    oa, op_, on = _normalize(vw, vi)

    anchor = oa.transpose(2, 0, 1)
    positive = op_.transpose(2, 0, 1)
    negative = (on.reshape(seq, batch // 128, num_neg, 128)
                .transpose(1, 3, 2, 0).reshape(batch, num_neg, seq))
    return (anchor, positive, negative)


# speedup vs baseline: 3.5147x; 3.5147x over previous
import functools

import jax
import jax.numpy as jnp
from jax import lax
from jax.experimental import pallas as pl
from jax.experimental.pallas import tpu as pltpu
from jax.experimental.pallas import tpu_sc as plsc

_DIM = 32
_MBLK = 65536


def _row_mean_body(x_ref, o_ref):
    o_ref[...] = jnp.sum(x_ref[...], axis=0) * (1.0 / _DIM)


def _row_means(table_t):
    rows = table_t.shape[1]
    grid = (rows + _MBLK - 1) // _MBLK
    return pl.pallas_call(
        _row_mean_body,
        grid=(grid,),
        in_specs=[pl.BlockSpec((_DIM, _MBLK), lambda i: (0, i))],
        out_specs=pl.BlockSpec((_MBLK,), lambda i: (i,)),
        out_shape=jax.ShapeDtypeStruct((grid * _MBLK,), jnp.float32),
    )(table_t)


def _gather_means(means, idx1d):
    info = plsc.get_sparse_core_info()
    nw = info.num_cores * info.num_subcores
    n = idx1d.shape[0]
    npw = n // nw
    mesh = plsc.VectorSubcoreMesh(core_axis_name="c", subcore_axis_name="s")

    @functools.partial(
        pl.kernel, mesh=mesh,
        out_type=jax.ShapeDtypeStruct((n,), jnp.float32),
        scratch_types=[
            pltpu.VMEM((npw,), jnp.int32),
            pltpu.VMEM((npw,), jnp.float32),
            pltpu.SemaphoreType.DMA,
        ],
    )
    def gather_kernel(means_hbm, idx_hbm, out_hbm, idx_v, vals_v, sem):
        wid = lax.axis_index("s") * info.num_cores + lax.axis_index("c")
        base = wid * npw
        pltpu.sync_copy(idx_hbm.at[pl.ds(base, npw)], idx_v)
        pltpu.async_copy(means_hbm.at[idx_v], vals_v, sem).wait()
        pltpu.sync_copy(vals_v, out_hbm.at[pl.ds(base, npw)])

    return gather_kernel(means, idx1d)


def _norm_body(a_ref, p_ref, n_ref, oa_ref, op_ref, on_ref):
    for x_ref, o_ref in ((a_ref, oa_ref), (p_ref, op_ref), (n_ref, on_ref)):
        x = x_ref[...]
        ss = jnp.sum(x * x, axis=0, keepdims=True)
        o_ref[...] = x / jnp.sqrt(ss)


def _normalize(va, vp, vn):
    seq = va.shape[0]
    ca, cn = va.shape[2], vn.shape[2]
    grid = 8
    ba, bn = ca // grid, cn // grid
    spec_a = pl.BlockSpec((seq, 1, ba), lambda i: (0, 0, i))
    spec_n = pl.BlockSpec((seq, 1, bn), lambda i: (0, 0, i))
    return pl.pallas_call(
        _norm_body,
        grid=(grid,),
        in_specs=[spec_a, spec_a, spec_n],
        out_specs=[spec_a, spec_a, spec_n],
        out_shape=[jax.ShapeDtypeStruct((seq, 1, ca), jnp.float32),
                   jax.ShapeDtypeStruct((seq, 1, ca), jnp.float32),
                   jax.ShapeDtypeStruct((seq, 1, cn), jnp.float32)],
    )(va, vp, vn)


def kernel(anchor_input_ids, positive_input_ids, negative_input_ids,
           embedding_table):
    batch, seq = anchor_input_ids.shape
    num_neg = negative_input_ids.shape[1]
    na = batch * seq

    means = _row_means(embedding_table.T)
    nt = negative_input_ids.transpose(2, 1, 0)
    nt = nt.reshape(seq, num_neg, batch // 128, 128).transpose(0, 2, 1, 3)
    ids = jnp.concatenate([
        anchor_input_ids.T.reshape(-1),
        positive_input_ids.T.reshape(-1),
        nt.reshape(-1),
    ]).astype(jnp.int32)
    vals = _gather_means(means, ids)

    va = vals[:na].reshape(seq, 1, batch)
    vp = vals[na:2 * na].reshape(seq, 1, batch)
    vn = vals[2 * na:].reshape(seq, 1, num_neg * batch)
    oa, op_, on = _normalize(va, vp, vn)

    anchor = oa.transpose(2, 0, 1)
    positive = op_.transpose(2, 0, 1)
    negative = (on.reshape(seq, batch // 128, num_neg, 128)
                .transpose(1, 3, 2, 0).reshape(batch, num_neg, seq))
    return (anchor, positive, negative)


# speedup vs baseline: 3.8171x; 1.0860x over previous
import functools

import jax
import jax.numpy as jnp
from jax import lax
from jax.experimental import pallas as pl
from jax.experimental.pallas import tpu as pltpu
from jax.experimental.pallas import tpu_sc as plsc

_DIM = 32
_MBLK = 65536


def _row_mean_body(x_ref, o_ref):
    o_ref[...] = jnp.sum(x_ref[...], axis=0) * (1.0 / _DIM)


def _row_means(table_t):
    rows = table_t.shape[1]
    grid = (rows + _MBLK - 1) // _MBLK
    return pl.pallas_call(
        _row_mean_body,
        grid=(grid,),
        in_specs=[pl.BlockSpec((_DIM, _MBLK), lambda i: (0, i))],
        out_specs=pl.BlockSpec((_MBLK,), lambda i: (i,)),
        out_shape=jax.ShapeDtypeStruct((grid * _MBLK,), jnp.float32),
    )(table_t)


def _gather_means(means, idx_a, idx_p, idx_n):
    info = plsc.get_sparse_core_info()
    nw = info.num_cores * info.num_subcores
    na, nn = idx_a.shape[0], idx_n.shape[0]
    apw, npw = na // nw, nn // nw
    tpw = 2 * apw + npw
    mesh = plsc.VectorSubcoreMesh(core_axis_name="c", subcore_axis_name="s")

    @functools.partial(
        pl.kernel, mesh=mesh,
        out_type=[jax.ShapeDtypeStruct((na,), jnp.float32),
                  jax.ShapeDtypeStruct((na,), jnp.float32),
                  jax.ShapeDtypeStruct((nn,), jnp.float32)],
        scratch_types=[
            pltpu.VMEM((tpw,), jnp.int32),
            pltpu.VMEM((tpw,), jnp.float32),
            pltpu.SemaphoreType.DMA,
        ],
    )
    def gather_kernel(means_hbm, a_hbm, p_hbm, n_hbm,
                      oa_hbm, op_hbm, on_hbm, idx_v, vals_v, sem):
        wid = lax.axis_index("s") * info.num_cores + lax.axis_index("c")
        ab, nb = wid * apw, wid * npw
        pltpu.sync_copy(a_hbm.at[pl.ds(ab, apw)], idx_v.at[pl.ds(0, apw)])
        pltpu.sync_copy(p_hbm.at[pl.ds(ab, apw)],
                        idx_v.at[pl.ds(apw, apw)])
        pltpu.sync_copy(n_hbm.at[pl.ds(nb, npw)],
                        idx_v.at[pl.ds(2 * apw, npw)])
        pltpu.async_copy(means_hbm.at[idx_v], vals_v, sem).wait()
        pltpu.sync_copy(vals_v.at[pl.ds(0, apw)], oa_hbm.at[pl.ds(ab, apw)])
        pltpu.sync_copy(vals_v.at[pl.ds(apw, apw)],
                        op_hbm.at[pl.ds(ab, apw)])
        pltpu.sync_copy(vals_v.at[pl.ds(2 * apw, npw)],
                        on_hbm.at[pl.ds(nb, npw)])

    return gather_kernel(means, idx_a, idx_p, idx_n)


def _norm_body(a_ref, p_ref, n_ref, oa_ref, op_ref, on_ref):
    for x_ref, o_ref in ((a_ref, oa_ref), (p_ref, op_ref), (n_ref, on_ref)):
        x = x_ref[...]
        ss = jnp.sum(x * x, axis=0, keepdims=True)
        o_ref[...] = x / jnp.sqrt(ss)


def _normalize(va, vp, vn):
    seq = va.shape[0]
    ca, cn = va.shape[2], vn.shape[2]
    grid = 8
    ba, bn = ca // grid, cn // grid
    spec_a = pl.BlockSpec((seq, 1, ba), lambda i: (0, 0, i))
    spec_n = pl.BlockSpec((seq, 1, bn), lambda i: (0, 0, i))
    return pl.pallas_call(
        _norm_body,
        grid=(grid,),
        in_specs=[spec_a, spec_a, spec_n],
        out_specs=[spec_a, spec_a, spec_n],
        out_shape=[jax.ShapeDtypeStruct((seq, 1, ca), jnp.float32),
                   jax.ShapeDtypeStruct((seq, 1, ca), jnp.float32),
                   jax.ShapeDtypeStruct((seq, 1, cn), jnp.float32)],
    )(va, vp, vn)


def kernel(anchor_input_ids, positive_input_ids, negative_input_ids,
           embedding_table):
    batch, seq = anchor_input_ids.shape
    num_neg = negative_input_ids.shape[1]
    na = batch * seq

    means = _row_means(embedding_table.T)
    nt = negative_input_ids.transpose(2, 1, 0)
    nt = nt.reshape(seq, num_neg, batch // 128, 128).transpose(0, 2, 1, 3)
    fa, fp, fn = _gather_means(means,
                               anchor_input_ids.T.reshape(-1),
                               positive_input_ids.T.reshape(-1),
                               nt.reshape(-1))

    va = fa.reshape(seq, 1, batch)
    vp = fp.reshape(seq, 1, batch)
    vn = fn.reshape(seq, 1, num_neg * batch)
    oa, op_, on = _normalize(va, vp, vn)

    anchor = oa.transpose(2, 0, 1)
    positive = op_.transpose(2, 0, 1)
    negative = (on.reshape(seq, batch // 128, num_neg, 128)
                .transpose(1, 3, 2, 0).reshape(batch, num_neg, seq))
    return (anchor, positive, negative)
